# baseline (speedup 1.0000x reference)
import functools

import jax
import jax.numpy as jnp
from jax import lax
from jax.experimental import pallas as pl
from jax.experimental.pallas import tpu as pltpu
from jax.experimental.pallas import tpu_sc as plsc

NUM_CORES = 2
NUM_SUBCORES = 16
NUM_WORKERS = NUM_CORES * NUM_SUBCORES
CHUNK_ROWS = 32


def kernel(tokens, embed_table):
    batch = tokens.shape[0]
    seq_len = tokens.shape[1]
    embed_dim = embed_table.shape[1]
    rows_per_worker = seq_len // NUM_WORKERS
    n_chunks = rows_per_worker // CHUNK_ROWS
    mesh = plsc.VectorSubcoreMesh(core_axis_name="c", subcore_axis_name="s")

    @functools.partial(
        pl.kernel,
        mesh=mesh,
        out_type=jax.ShapeDtypeStruct(
            (batch, seq_len, embed_dim), embed_table.dtype),
        scratch_types=[
            pltpu.VMEM((CHUNK_ROWS, embed_dim), jnp.float32),
            pltpu.VMEM((CHUNK_ROWS, embed_dim), jnp.float32),
            pltpu.SemaphoreType.DMA,
            pltpu.SemaphoreType.DMA,
            pltpu.SemaphoreType.DMA,
            pltpu.SemaphoreType.DMA,
        ],
    )
    def sc_copy(table_hbm, out_hbm, buf0, buf1, rsem0, rsem1, wsem0, wsem1):
        wid = lax.axis_index("s") * NUM_CORES + lax.axis_index("c")
        base = wid * rows_per_worker
        bufs = (buf0, buf1)
        rsems = (rsem0, rsem1)
        wsems = (wsem0, wsem1)

        def read(i):
            s = i % 2
            return pltpu.async_copy(
                table_hbm.at[pl.ds(base + i * CHUNK_ROWS, CHUNK_ROWS)],
                bufs[s], rsems[s])

        def writes(i):
            s = i % 2
            return [
                pltpu.async_copy(
                    bufs[s],
                    out_hbm.at[b, pl.ds(base + i * CHUNK_ROWS, CHUNK_ROWS)],
                    wsems[s])
                for b in range(batch)
            ]

        pending_reads = {0: read(0)}
        pending_writes = {}
        for i in range(n_chunks):
            pending_reads.pop(i).wait()
            pending_writes[i] = writes(i)
            if i + 1 < n_chunks:
                if i >= 1:
                    for h in pending_writes.pop(i - 1):
                        h.wait()
                pending_reads[i + 1] = read(i + 1)
        for i in sorted(pending_writes):
            for h in pending_writes.pop(i):
                h.wait()

    return sc_copy(embed_table[:seq_len])


# speedup vs baseline: 1.0601x; 1.0601x over previous
import functools

import jax
import jax.numpy as jnp
from jax import lax
from jax.experimental import pallas as pl
from jax.experimental.pallas import tpu as pltpu
from jax.experimental.pallas import tpu_sc as plsc

NUM_CORES = 2
NUM_SUBCORES = 16
NUM_WORKERS = NUM_CORES * NUM_SUBCORES
CHUNK_ROWS = 64


def kernel(tokens, embed_table):
    batch = tokens.shape[0]
    seq_len = tokens.shape[1]
    embed_dim = embed_table.shape[1]
    rows_per_worker = seq_len // NUM_WORKERS
    n_chunks = rows_per_worker // CHUNK_ROWS
    mesh = plsc.VectorSubcoreMesh(core_axis_name="c", subcore_axis_name="s")

    @functools.partial(
        pl.kernel,
        mesh=mesh,
        out_type=jax.ShapeDtypeStruct(
            (batch, seq_len, embed_dim), embed_table.dtype),
        scratch_types=[
            pltpu.VMEM((CHUNK_ROWS, embed_dim), jnp.float32),
            pltpu.SemaphoreType.DMA,
        ],
    )
    def sc_copy(table_hbm, out_hbm, buf, wsem):
        wid = lax.axis_index("s") * NUM_CORES + lax.axis_index("c")
        base = wid * rows_per_worker

        for i in range(n_chunks):
            r = base + i * CHUNK_ROWS
            pltpu.sync_copy(table_hbm.at[pl.ds(r, CHUNK_ROWS)], buf)
            handles = [
                pltpu.async_copy(
                    buf, out_hbm.at[b, pl.ds(r, CHUNK_ROWS)], wsem)
                for b in range(batch)
            ]
            for h in handles:
                h.wait()

    return sc_copy(embed_table[:seq_len])
